# baseline (speedup 1.0000x reference)
import dataclasses
import functools

import jax
import jax.numpy as jnp
import numpy as np
from jax import lax
from jax.experimental import pallas as pl
from jax.experimental.pallas import tpu as pltpu
from jax.experimental.pallas import tpu_sc as plsc

NUM_NEG = 128
MARGIN = 0.5
INT_MIN = np.int32(-2147483648)
INT_MAX = np.int32(2147483647)

B_TOT = 1024
S_SC = 128
B_TC = B_TOT - S_SC
NC, NS, L = 2, 16, 16
NW = NC * NS
PER_W = S_SC // NW


def _f32_key(bits):
    return jnp.where(bits >= 0, bits, INT_MIN - bits)


def _prep_body(qs_ref, c_ref, l_ref, t_ref):
    qs = qs_ref[...]
    c = c_ref[...]
    l = jax.lax.dot_general(
        qs, c, (((1,), (1,)), ((), ())),
        preferred_element_type=jnp.float32,
        precision=jax.lax.Precision.HIGHEST,
    )
    S, C = l.shape
    rows = jax.lax.broadcasted_iota(jnp.int32, (S, C), 0)
    cols = jax.lax.broadcasted_iota(jnp.int32, (S, C), 1)
    eye = cols == rows + B_TC
    t = jnp.sum(jnp.where(eye, l, 0.0), axis=1, keepdims=True)
    l_ref[...] = l
    t_ref[...] = jnp.broadcast_to(t, (S, L))


def _tc_body(q_ref, c_ref, out_ref):
    q = q_ref[...]
    c = c_ref[...]
    lt = jax.lax.dot_general(
        c, q, (((1,), (1,)), ((), ())),
        preferred_element_type=jnp.float32,
        precision=jax.lax.Precision.HIGHEST,
    )

    C, B = lt.shape
    rows = jax.lax.broadcasted_iota(jnp.int32, (C, B), 0)
    cols = jax.lax.broadcasted_iota(jnp.int32, (C, B), 1)
    eye = rows == cols
    t = jnp.sum(jnp.where(eye, lt, 0.0), axis=0, keepdims=True)

    bits = jax.lax.bitcast_convert_type(lt, jnp.int32)
    key = _f32_key(bits)
    tkey = _f32_key(jax.lax.bitcast_convert_type(t, jnp.int32))
    neg = key < tkey
    mk = jnp.where(neg, key, INT_MIN)

    n = jnp.sum(neg.astype(jnp.int32), axis=0, keepdims=True)
    k = jnp.minimum(n, NUM_NEG)
    kk = jnp.maximum(k, 1)

    def cond(carry):
        lo, hi = carry
        return jnp.any((hi - lo) != 1)

    def step(carry):
        lo, hi = carry
        mid = (lo >> 1) + (hi >> 1) + (lo & hi & 1)
        cnt = jnp.sum((mk >= mid).astype(jnp.int32), axis=0, keepdims=True)
        pred = cnt >= kk
        return jnp.where(pred, mid, lo), jnp.where(pred, hi, mid)

    lo0 = jnp.full((1, B), INT_MIN, jnp.int32)
    hi0 = jnp.full((1, B), INT_MAX, jnp.int32)
    tau, _ = jax.lax.while_loop(cond, step, (lo0, hi0))

    above = mk > tau
    c_gt = jnp.sum(above.astype(jnp.int32), axis=0, keepdims=True)
    m = (k - c_gt).astype(jnp.float32)
    tau_f = jax.lax.bitcast_convert_type(_f32_key(tau), jnp.float32)

    relu = jnp.maximum(lt - (1.0 - MARGIN), 0.0)
    num = jnp.sum(jnp.where(above, relu, 0.0), axis=0, keepdims=True)
    num = num + m * jnp.maximum(tau_f - (1.0 - MARGIN), 0.0)
    num = jnp.where(k > 0, num, 0.0)
    contrastive = num / (k.astype(jnp.float32) + 1e-9)

    align = jnp.sum(1.0 - t, axis=(0, 1), keepdims=True)
    out_ref[...] = align + jnp.sum(contrastive, axis=(0, 1), keepdims=True)


_NSLICE = B_TOT // L
_NACC = 8


def _lane_sum_i32(parts):
    tot = parts[0]
    for p in parts[1:]:
        tot = tot + p
    return jnp.sum(tot)


def _sc_process_query(row_ref, mk_v, t_ref, acc_v):
    t16 = t_ref[...]

    nparts = [jnp.zeros((L,), jnp.int32) for _ in range(_NACC)]
    one = jnp.ones((L,), jnp.int32)
    zero = jnp.zeros((L,), jnp.int32)
    for j in range(_NSLICE):
        v = row_ref[pl.ds(j * L, L)]
        bits = plsc.bitcast(v, jnp.int32)
        neg = v < t16
        key = jnp.where(bits >= 0, bits, INT_MIN - bits)
        mk_v[pl.ds(j * L, L)] = jnp.where(neg, key, INT_MIN)
        a = j % _NACC
        nparts[a] = nparts[a] + jnp.where(neg, one, zero)
    n = _lane_sum_i32(nparts)
    k = jnp.minimum(n, NUM_NEG)
    kk = jnp.maximum(k, 1)

    def cond(carry):
        lo, hi = carry
        return (hi - lo) != 1

    def step(carry):
        lo, hi = carry
        mid = (lo >> 1) + (hi >> 1) + (lo & hi & 1)
        mid_v = jnp.broadcast_to(mid, (L,))
        parts = [jnp.zeros((L,), jnp.int32) for _ in range(_NACC)]
        for j in range(_NSLICE):
            m = mk_v[pl.ds(j * L, L)]
            a = j % _NACC
            parts[a] = parts[a] + jnp.where(m >= mid_v, one, zero)
        cnt = _lane_sum_i32(parts)
        pred = cnt >= kk
        return jnp.where(pred, mid, lo), jnp.where(pred, hi, mid)

    tau, _ = lax.while_loop(cond, step, (INT_MIN + 0, INT_MAX + 0))

    tau_v = jnp.broadcast_to(tau, (L,))
    gparts = [jnp.zeros((L,), jnp.int32) for _ in range(_NACC)]
    sparts = [jnp.zeros((L,), jnp.float32) for _ in range(_NACC)]
    fzero = jnp.zeros((L,), jnp.float32)
    for j in range(_NSLICE):
        m = mk_v[pl.ds(j * L, L)]
        v = row_ref[pl.ds(j * L, L)]
        ab = m > tau_v
        a = j % _NACC
        gparts[a] = gparts[a] + jnp.where(ab, one, zero)
        sparts[a] = sparts[a] + jnp.where(
            ab, jnp.maximum(v - (1.0 - MARGIN), 0.0), fzero)
    c_gt = _lane_sum_i32(gparts)
    stot = sparts[0]
    for p in sparts[1:]:
        stot = stot + p
    s_sum = jnp.sum(stot)

    m_tie = (k - c_gt).astype(jnp.float32)
    tau_f = plsc.bitcast(
        jnp.broadcast_to(jnp.where(tau >= 0, tau, INT_MIN - tau), (L,)),
        jnp.float32)
    num = s_sum + m_tie * jnp.maximum(tau_f - (1.0 - MARGIN), 0.0)
    num = jnp.where(n > 0, num, 0.0)
    contr = num / (k.astype(jnp.float32) + 1e-9)
    acc_v[...] = acc_v[...] + contr + (1.0 - t16)


def _sc_body(l_hbm, t_hbm, out_hbm, row_a, row_b, mk_v, t_a, t_b, acc_v,
             sem0, sem1):
    wid = lax.axis_index("s") * NC + lax.axis_index("c")
    base = wid * PER_W
    acc_v[...] = jnp.zeros((L,), jnp.float32)
    rows = (row_a, row_b)
    ts = (t_a, t_b)
    sems = (sem0, sem1)

    copies = {}
    for qi in range(min(1, PER_W)):
        b = qi % 2
        copies[qi] = (
            pltpu.async_copy(l_hbm.at[base + qi], rows[b], sems[b]),
            pltpu.async_copy(t_hbm.at[base + qi], ts[b], sems[b]),
        )
    for qi in range(PER_W):
        b = qi % 2
        for h in copies.pop(qi):
            h.wait()
        if qi + 1 < PER_W:
            nb = (qi + 1) % 2
            copies[qi + 1] = (
                pltpu.async_copy(l_hbm.at[base + qi + 1], rows[nb],
                                 sems[nb]),
                pltpu.async_copy(t_hbm.at[base + qi + 1], ts[nb],
                                 sems[nb]),
            )
        _sc_process_query(rows[b], mk_v, ts[b], acc_v)

    pltpu.sync_copy(acc_v, out_hbm.at[wid])


_sc_mesh = plsc.VectorSubcoreMesh(core_axis_name="c", subcore_axis_name="s")

_sc_cp = pltpu.CompilerParams()
if "needs_layout_passes" in pltpu.CompilerParams.__dataclass_fields__:
    _sc_cp = dataclasses.replace(_sc_cp, needs_layout_passes=False)

_sc_kernel = functools.partial(
    pl.kernel,
    mesh=_sc_mesh,
    compiler_params=_sc_cp,
    out_type=jax.ShapeDtypeStruct((NW, L), jnp.float32),
    scratch_types=[
        pltpu.VMEM((B_TOT,), jnp.float32),
        pltpu.VMEM((B_TOT,), jnp.float32),
        pltpu.VMEM((B_TOT,), jnp.int32),
        pltpu.VMEM((L,), jnp.float32),
        pltpu.VMEM((L,), jnp.float32),
        pltpu.VMEM((L,), jnp.float32),
        pltpu.SemaphoreType.DMA,
        pltpu.SemaphoreType.DMA,
    ],
)(_sc_body)


@jax.jit
def kernel(query_embed, candidate_embed):
    q = query_embed.reshape(query_embed.shape[0], query_embed.shape[2])
    c = candidate_embed.reshape(candidate_embed.shape[1],
                                candidate_embed.shape[2])
    l_sc, t_sc = pl.pallas_call(
        _prep_body,
        out_shape=(
            jax.ShapeDtypeStruct((S_SC, B_TOT), jnp.float32),
            jax.ShapeDtypeStruct((S_SC, L), jnp.float32),
        ),
    )(q[B_TC:], c)
    sc_part = _sc_kernel(l_sc, t_sc)
    tc_part = pl.pallas_call(
        _tc_body,
        out_shape=jax.ShapeDtypeStruct((1, 1), jnp.float32),
    )(q[:B_TC], c)
    return tc_part[0, 0] + jnp.sum(sc_part[:, 0])


# speedup vs baseline: 1.0045x; 1.0045x over previous
import dataclasses
import functools

import jax
import jax.numpy as jnp
import numpy as np
from jax import lax
from jax.experimental import pallas as pl
from jax.experimental.pallas import tpu as pltpu
from jax.experimental.pallas import tpu_sc as plsc

NUM_NEG = 128
MARGIN = 0.5
INT_MIN = np.int32(-2147483648)
INT_MAX = np.int32(2147483647)

B_TOT = 1024
S_SC = 128
B_TC = B_TOT - S_SC
NC, NS, L = 2, 16, 16
NW = NC * NS
PER_W = S_SC // NW


def _f32_key(bits):
    return jnp.where(bits >= 0, bits, INT_MIN - bits)


def _prep_body(qs_ref, c_ref, l_ref, t_ref):
    qs = qs_ref[...]
    c = c_ref[...]
    l = jax.lax.dot_general(
        qs, c, (((1,), (1,)), ((), ())),
        preferred_element_type=jnp.float32,
        precision=jax.lax.Precision.HIGHEST,
    )
    S, C = l.shape
    rows = jax.lax.broadcasted_iota(jnp.int32, (S, C), 0)
    cols = jax.lax.broadcasted_iota(jnp.int32, (S, C), 1)
    eye = cols == rows + B_TC
    t = jnp.sum(jnp.where(eye, l, 0.0), axis=1, keepdims=True)
    l_ref[...] = l
    t_ref[...] = jnp.broadcast_to(t, (S, L))


def _tc_body(q_ref, c_ref, out_ref):
    q = q_ref[...]
    c = c_ref[...]
    lt = jax.lax.dot_general(
        c, q, (((1,), (1,)), ((), ())),
        preferred_element_type=jnp.float32,
        precision=jax.lax.Precision.HIGHEST,
    )

    C, B = lt.shape
    rows = jax.lax.broadcasted_iota(jnp.int32, (C, B), 0)
    cols = jax.lax.broadcasted_iota(jnp.int32, (C, B), 1)
    eye = rows == cols
    t = jnp.sum(jnp.where(eye, lt, 0.0), axis=0, keepdims=True)

    bits = jax.lax.bitcast_convert_type(lt, jnp.int32)
    key = _f32_key(bits)
    tkey = _f32_key(jax.lax.bitcast_convert_type(t, jnp.int32))
    neg = key < tkey
    mk = jnp.where(neg, key, INT_MIN)

    n = jnp.sum(neg.astype(jnp.int32), axis=0, keepdims=True)
    k = jnp.minimum(n, NUM_NEG)
    kk = jnp.maximum(k, 1)

    def cond(carry):
        lo, hi = carry
        return jnp.any((hi - lo) != 1)

    def step(carry):
        lo, hi = carry
        mid = (lo >> 1) + (hi >> 1) + (lo & hi & 1)
        cnt = jnp.sum((mk >= mid).astype(jnp.int32), axis=0, keepdims=True)
        pred = cnt >= kk
        return jnp.where(pred, mid, lo), jnp.where(pred, hi, mid)

    lo0 = jnp.full((1, B), INT_MIN, jnp.int32)
    hi0 = jnp.full((1, B), INT_MAX, jnp.int32)
    tau, _ = jax.lax.while_loop(cond, step, (lo0, hi0))

    above = mk > tau
    c_gt = jnp.sum(above.astype(jnp.int32), axis=0, keepdims=True)
    m = (k - c_gt).astype(jnp.float32)
    tau_f = jax.lax.bitcast_convert_type(_f32_key(tau), jnp.float32)

    relu = jnp.maximum(lt - (1.0 - MARGIN), 0.0)
    num = jnp.sum(jnp.where(above, relu, 0.0), axis=0, keepdims=True)
    num = num + m * jnp.maximum(tau_f - (1.0 - MARGIN), 0.0)
    num = jnp.where(k > 0, num, 0.0)
    contrastive = num / (k.astype(jnp.float32) + 1e-9)

    align = jnp.sum(1.0 - t, axis=(0, 1), keepdims=True)
    out_ref[...] = align + jnp.sum(contrastive, axis=(0, 1), keepdims=True)


_NSLICE = B_TOT // L
_NACC = 8


def _lane_sum_i32(parts):
    tot = parts[0]
    for p in parts[1:]:
        tot = tot + p
    return jnp.sum(tot)


def _sc_process_query(row_ref, mk_v, t_ref, acc_v):
    t16 = t_ref[...]

    nparts = [jnp.zeros((L,), jnp.int32) for _ in range(_NACC)]
    one = jnp.ones((L,), jnp.int32)
    zero = jnp.zeros((L,), jnp.int32)
    for j in range(_NSLICE):
        v = row_ref[pl.ds(j * L, L)]
        bits = plsc.bitcast(v, jnp.int32)
        neg = v < t16
        key = jnp.where(bits >= 0, bits, INT_MIN - bits)
        mk_v[pl.ds(j * L, L)] = jnp.where(neg, key, INT_MIN)
        a = j % _NACC
        nparts[a] = nparts[a] + jnp.where(neg, one, zero)
    n = _lane_sum_i32(nparts)
    k = jnp.minimum(n, NUM_NEG)
    kk = jnp.maximum(k, 1)

    def cond(carry):
        lo, hi = carry
        return (hi - lo) != 1

    def step(carry):
        lo, hi = carry
        mid = (lo >> 1) + (hi >> 1) + (lo & hi & 1)
        mid_v = jnp.broadcast_to(mid, (L,))
        parts = [jnp.zeros((L,), jnp.int32) for _ in range(_NACC)]
        for j in range(_NSLICE):
            m = mk_v[pl.ds(j * L, L)]
            a = j % _NACC
            parts[a] = parts[a] + jnp.where(m >= mid_v, one, zero)
        cnt = _lane_sum_i32(parts)
        pred = cnt >= kk
        return jnp.where(pred, mid, lo), jnp.where(pred, hi, mid)

    tau, _ = lax.while_loop(cond, step, (INT_MIN + 0, INT_MAX + 0))

    tau_v = jnp.broadcast_to(tau, (L,))
    gparts = [jnp.zeros((L,), jnp.int32) for _ in range(_NACC)]
    sparts = [jnp.zeros((L,), jnp.float32) for _ in range(_NACC)]
    fzero = jnp.zeros((L,), jnp.float32)
    for j in range(_NSLICE):
        m = mk_v[pl.ds(j * L, L)]
        v = row_ref[pl.ds(j * L, L)]
        ab = m > tau_v
        a = j % _NACC
        gparts[a] = gparts[a] + jnp.where(ab, one, zero)
        sparts[a] = sparts[a] + jnp.where(
            ab, jnp.maximum(v - (1.0 - MARGIN), 0.0), fzero)
    c_gt = _lane_sum_i32(gparts)
    stot = sparts[0]
    for p in sparts[1:]:
        stot = stot + p
    s_sum = jnp.sum(stot)

    m_tie = (k - c_gt).astype(jnp.float32)
    tau_f = plsc.bitcast(
        jnp.broadcast_to(jnp.where(tau >= 0, tau, INT_MIN - tau), (L,)),
        jnp.float32)
    num = s_sum + m_tie * jnp.maximum(tau_f - (1.0 - MARGIN), 0.0)
    num = jnp.where(n > 0, num, 0.0)
    contr = num / (k.astype(jnp.float32) + 1e-9)
    acc_v[...] = acc_v[...] + contr + (1.0 - t16)


def _sc_body(l_hbm, t_hbm, out_hbm, row_a, row_b, mk_v, t_a, t_b, acc_v,
             sem0, sem1):
    wid = lax.axis_index("s") * NC + lax.axis_index("c")
    base = wid * PER_W
    acc_v[...] = jnp.zeros((L,), jnp.float32)
    rows = (row_a, row_b)
    ts = (t_a, t_b)
    sems = (sem0, sem1)

    copies = {}
    for qi in range(min(1, PER_W)):
        b = qi % 2
        copies[qi] = (
            pltpu.async_copy(l_hbm.at[base + qi], rows[b], sems[b]),
            pltpu.async_copy(t_hbm.at[base + qi], ts[b], sems[b]),
        )
    for qi in range(PER_W):
        b = qi % 2
        for h in copies.pop(qi):
            h.wait()
        if qi + 1 < PER_W:
            nb = (qi + 1) % 2
            copies[qi + 1] = (
                pltpu.async_copy(l_hbm.at[base + qi + 1], rows[nb],
                                 sems[nb]),
                pltpu.async_copy(t_hbm.at[base + qi + 1], ts[nb],
                                 sems[nb]),
            )
        _sc_process_query(rows[b], mk_v, ts[b], acc_v)

    pltpu.sync_copy(acc_v, out_hbm.at[wid])


_sc_mesh = plsc.VectorSubcoreMesh(core_axis_name="c", subcore_axis_name="s")

_sc_cp = pltpu.CompilerParams()
if "needs_layout_passes" in pltpu.CompilerParams.__dataclass_fields__:
    _sc_cp = dataclasses.replace(_sc_cp, needs_layout_passes=False)

_sc_kernel = functools.partial(
    pl.kernel,
    mesh=_sc_mesh,
    compiler_params=_sc_cp,
    out_type=jax.ShapeDtypeStruct((NW, L), jnp.float32),
    scratch_types=[
        pltpu.VMEM((B_TOT,), jnp.float32),
        pltpu.VMEM((B_TOT,), jnp.float32),
        pltpu.VMEM((B_TOT,), jnp.int32),
        pltpu.VMEM((L,), jnp.float32),
        pltpu.VMEM((L,), jnp.float32),
        pltpu.VMEM((L,), jnp.float32),
        pltpu.SemaphoreType.DMA,
        pltpu.SemaphoreType.DMA,
    ],
)(_sc_body)


@jax.jit
def kernel(query_embed, candidate_embed):
    q = query_embed.reshape(query_embed.shape[0], query_embed.shape[2])
    c = candidate_embed.reshape(candidate_embed.shape[1],
                                candidate_embed.shape[2])
    l_sc, t_sc = pl.pallas_call(
        _prep_body,
        out_shape=(
            jax.ShapeDtypeStruct((S_SC, B_TOT), jnp.float32),
            jax.ShapeDtypeStruct((S_SC, L), jnp.float32),
        ),
    )(q[B_TC:], c)
    tc_part = pl.pallas_call(
        _tc_body,
        out_shape=jax.ShapeDtypeStruct((1, 1), jnp.float32),
    )(q[:B_TC], c)
    sc_part = _sc_kernel(l_sc, t_sc)
    return tc_part[0, 0] + jnp.sum(sc_part[:, 0])


# speedup vs baseline: 1.5653x; 1.5584x over previous
import functools

import jax
import jax.numpy as jnp
import numpy as np
from jax.experimental import pallas as pl
from jax.experimental.pallas import tpu as pltpu

NUM_NEG = 128
MARGIN = 0.5
INT_MIN = np.int32(-2147483648)
INT_MAX = np.int32(2147483647)


def _f32_key(bits):
    return jnp.where(bits >= 0, bits, INT_MIN - bits)


def _loss_body(q_ref, c_ref, out_ref):
    q = q_ref[...]
    c = c_ref[...]
    lt = jax.lax.dot_general(
        c, q, (((1,), (1,)), ((), ())),
        preferred_element_type=jnp.float32,
        precision=jax.lax.Precision.HIGHEST,
    )

    C, B = lt.shape
    rows = jax.lax.broadcasted_iota(jnp.int32, (C, B), 0)
    cols = jax.lax.broadcasted_iota(jnp.int32, (C, B), 1)
    eye = rows == cols
    t = jnp.sum(jnp.where(eye, lt, 0.0), axis=0, keepdims=True)

    bits = jax.lax.bitcast_convert_type(lt, jnp.int32)
    key = _f32_key(bits)
    tkey = _f32_key(jax.lax.bitcast_convert_type(t, jnp.int32))
    neg = key < tkey
    mk = jnp.where(neg, key, INT_MIN)

    n = jnp.sum(neg.astype(jnp.int32), axis=0, keepdims=True)
    k = jnp.minimum(n, NUM_NEG)
    kk = jnp.maximum(k, 1)

    def _avg(a, b):
        return (a >> 1) + (b >> 1) + (a & b & 1)

    def cond(carry):
        lo, hi = carry
        return jnp.any((hi - lo) != 1)

    def step(carry):
        lo, hi = carry
        m2 = _avg(lo, hi)
        m1 = _avg(lo, m2)
        m3 = _avg(m2, hi)
        c1 = jnp.sum((mk >= m1).astype(jnp.int32), axis=0, keepdims=True)
        c2 = jnp.sum((mk >= m2).astype(jnp.int32), axis=0, keepdims=True)
        c3 = jnp.sum((mk >= m3).astype(jnp.int32), axis=0, keepdims=True)
        p2 = c2 >= kk
        lo1 = jnp.where(p2, m2, lo)
        hi1 = jnp.where(p2, hi, m2)
        ps = jnp.where(p2, c3, c1) >= kk
        ms = jnp.where(p2, m3, m1)
        lo2 = jnp.where(ps, ms, lo1)
        hi2 = jnp.where(ps, hi1, ms)
        return lo2, hi2

    lo0 = jnp.full((1, B), INT_MIN, jnp.int32)
    hi0 = jnp.full((1, B), INT_MAX, jnp.int32)
    tau, _ = jax.lax.while_loop(cond, step, (lo0, hi0))

    above = mk > tau
    c_gt = jnp.sum(above.astype(jnp.int32), axis=0, keepdims=True)
    m = (k - c_gt).astype(jnp.float32)
    tau_f = jax.lax.bitcast_convert_type(_f32_key(tau), jnp.float32)

    relu = jnp.maximum(lt - (1.0 - MARGIN), 0.0)
    num = jnp.sum(jnp.where(above, relu, 0.0), axis=0, keepdims=True)
    num = num + m * jnp.maximum(tau_f - (1.0 - MARGIN), 0.0)
    num = jnp.where(k > 0, num, 0.0)
    contrastive = num / (k.astype(jnp.float32) + 1e-9)

    align = jnp.sum(1.0 - t, axis=(0, 1), keepdims=True)
    out_ref[...] = align + jnp.sum(contrastive, axis=(0, 1), keepdims=True)


@jax.jit
def kernel(query_embed, candidate_embed):
    q = query_embed.reshape(query_embed.shape[0], query_embed.shape[2])
    c = candidate_embed.reshape(candidate_embed.shape[1],
                                candidate_embed.shape[2])
    out = pl.pallas_call(
        _loss_body,
        out_shape=jax.ShapeDtypeStruct((1, 1), jnp.float32),
    )(q, c)
    return out[0, 0]


# speedup vs baseline: 1.6780x; 1.0720x over previous
import functools

import jax
import jax.numpy as jnp
import numpy as np
from jax.experimental import pallas as pl
from jax.experimental.pallas import tpu as pltpu

NUM_NEG = 128
MARGIN = 0.5
INT_MIN = np.int32(-2147483648)
INT_MAX = np.int32(2147483647)


def _f32_key(bits):
    return jnp.where(bits >= 0, bits, INT_MIN - bits)


def _loss_body(q_ref, c_ref, out_ref):
    q = q_ref[...]
    c = c_ref[...]
    lt = jax.lax.dot_general(
        c, q, (((1,), (1,)), ((), ())),
        preferred_element_type=jnp.float32,
        precision=jax.lax.Precision.HIGHEST,
    )

    C, B = lt.shape
    rows = jax.lax.broadcasted_iota(jnp.int32, (C, B), 0)
    cols = jax.lax.broadcasted_iota(jnp.int32, (C, B), 1)
    eye = rows == cols
    t = jnp.sum(jnp.where(eye, lt, 0.0), axis=0, keepdims=True)

    bits = jax.lax.bitcast_convert_type(lt, jnp.int32)
    key = _f32_key(bits)
    tkey = _f32_key(jax.lax.bitcast_convert_type(t, jnp.int32))
    neg = key < tkey
    mk = jnp.where(neg, key, INT_MIN)

    n = jnp.sum(neg.astype(jnp.int32), axis=0, keepdims=True)
    k = jnp.minimum(n, NUM_NEG)
    kk = jnp.maximum(k, 1)

    def _avg(a, b):
        return (a >> 1) + (b >> 1) + (a & b & 1)

    def cond(carry):
        lo, hi = carry
        return jnp.any((hi - lo) != 1)

    W2 = np.int32(1 << 11)
    W3 = np.int32(1 << 22)
    MASK11 = np.int32(0x7FF)

    def step(carry):
        lo, hi = carry
        m2 = _avg(lo, hi)
        m1 = _avg(lo, m2)
        m3 = _avg(m2, hi)
        e = jnp.where(mk >= m2,
                      jnp.where(mk >= m3, W3, W2),
                      jnp.where(mk >= m1, np.int32(1), np.int32(0)))
        tot = jnp.sum(e, axis=0, keepdims=True)
        c3 = jax.lax.shift_right_logical(tot, 22)
        c2 = c3 + (jax.lax.shift_right_logical(tot, 11) & MASK11)
        c1 = c2 + (tot & MASK11)
        p2 = c2 >= kk
        lo1 = jnp.where(p2, m2, lo)
        hi1 = jnp.where(p2, hi, m2)
        ps = jnp.where(p2, c3, c1) >= kk
        ms = jnp.where(p2, m3, m1)
        lo2 = jnp.where(ps, ms, lo1)
        hi2 = jnp.where(ps, hi1, ms)
        return lo2, hi2

    lo0 = jnp.full((1, B), INT_MIN, jnp.int32)
    hi0 = jnp.full((1, B), INT_MAX, jnp.int32)
    tau, _ = jax.lax.while_loop(cond, step, (lo0, hi0))

    above = mk > tau
    c_gt = jnp.sum(above.astype(jnp.int32), axis=0, keepdims=True)
    m = (k - c_gt).astype(jnp.float32)
    tau_f = jax.lax.bitcast_convert_type(_f32_key(tau), jnp.float32)

    relu = jnp.maximum(lt - (1.0 - MARGIN), 0.0)
    num = jnp.sum(jnp.where(above, relu, 0.0), axis=0, keepdims=True)
    num = num + m * jnp.maximum(tau_f - (1.0 - MARGIN), 0.0)
    num = jnp.where(k > 0, num, 0.0)
    contrastive = num / (k.astype(jnp.float32) + 1e-9)

    align = jnp.sum(1.0 - t, axis=(0, 1), keepdims=True)
    out_ref[...] = align + jnp.sum(contrastive, axis=(0, 1), keepdims=True)


@jax.jit
def kernel(query_embed, candidate_embed):
    q = query_embed.reshape(query_embed.shape[0], query_embed.shape[2])
    c = candidate_embed.reshape(candidate_embed.shape[1],
                                candidate_embed.shape[2])
    out = pl.pallas_call(
        _loss_body,
        out_shape=jax.ShapeDtypeStruct((1, 1), jnp.float32),
    )(q, c)
    return out[0, 0]


# speedup vs baseline: 1.8084x; 1.0777x over previous
import functools

import jax
import jax.numpy as jnp
import numpy as np
from jax.experimental import pallas as pl
from jax.experimental.pallas import tpu as pltpu

NUM_NEG = 128
MARGIN = 0.5
INT_MIN = np.int32(-2147483648)
INT_MAX = np.int32(2147483647)


def _f32_key(bits):
    return jnp.where(bits >= 0, bits, INT_MIN - bits)


def _loss_body(q_ref, c_ref, out_ref):
    q = q_ref[...]
    c = c_ref[...]
    lt = jax.lax.dot_general(
        c, q, (((1,), (1,)), ((), ())),
        preferred_element_type=jnp.float32,
        precision=jax.lax.Precision.HIGHEST,
    )

    C, B = lt.shape
    rows = jax.lax.broadcasted_iota(jnp.int32, (C, B), 0)
    cols = jax.lax.broadcasted_iota(jnp.int32, (C, B), 1)
    eye = rows == cols
    t = jnp.sum(jnp.where(eye, lt, 0.0), axis=0, keepdims=True)

    bits = jax.lax.bitcast_convert_type(lt, jnp.int32)
    key = _f32_key(bits)
    tkey = _f32_key(jax.lax.bitcast_convert_type(t, jnp.int32))
    neg = key < tkey
    mk = jnp.where(neg, key, INT_MIN)

    n = jnp.sum(neg.astype(jnp.int32), axis=0, keepdims=True)
    k = jnp.minimum(n, NUM_NEG)
    kk = jnp.maximum(k, 1)

    def _avg(a, b):
        return (a >> 1) + (b >> 1) + (a & b & 1)

    W2 = np.int32(1 << 11)
    W3 = np.int32(1 << 22)
    MASK11 = np.int32(0x7FF)

    def step(_, carry):
        lo, hi = carry
        m2 = _avg(lo, hi)
        m1 = _avg(lo, m2)
        m3 = _avg(m2, hi)
        e = jnp.where(mk >= m2,
                      jnp.where(mk >= m3, W3, W2),
                      jnp.where(mk >= m1, np.int32(1), np.int32(0)))
        tot = jnp.sum(e, axis=0, keepdims=True)
        c3 = jax.lax.shift_right_logical(tot, 22)
        c2 = c3 + (jax.lax.shift_right_logical(tot, 11) & MASK11)
        c1 = c2 + (tot & MASK11)
        p2 = c2 >= kk
        lo1 = jnp.where(p2, m2, lo)
        hi1 = jnp.where(p2, hi, m2)
        ps = jnp.where(p2, c3, c1) >= kk
        ms = jnp.where(p2, m3, m1)
        lo2 = jnp.where(ps, ms, lo1)
        hi2 = jnp.where(ps, hi1, ms)
        return lo2, hi2

    lo0 = jnp.full((1, B), INT_MIN, jnp.int32)
    hi0 = jnp.full((1, B), INT_MAX, jnp.int32)
    tau, _ = jax.lax.fori_loop(0, 16, step, (lo0, hi0))

    above = mk > tau
    c_gt = jnp.sum(above.astype(jnp.int32), axis=0, keepdims=True)
    m = (k - c_gt).astype(jnp.float32)
    tau_f = jax.lax.bitcast_convert_type(_f32_key(tau), jnp.float32)

    relu = jnp.maximum(lt - (1.0 - MARGIN), 0.0)
    num = jnp.sum(jnp.where(above, relu, 0.0), axis=0, keepdims=True)
    num = num + m * jnp.maximum(tau_f - (1.0 - MARGIN), 0.0)
    num = jnp.where(k > 0, num, 0.0)
    contrastive = num / (k.astype(jnp.float32) + 1e-9)

    align = jnp.sum(1.0 - t, axis=(0, 1), keepdims=True)
    out_ref[...] = align + jnp.sum(contrastive, axis=(0, 1), keepdims=True)


@jax.jit
def kernel(query_embed, candidate_embed):
    q = query_embed.reshape(query_embed.shape[0], query_embed.shape[2])
    c = candidate_embed.reshape(candidate_embed.shape[1],
                                candidate_embed.shape[2])
    out = pl.pallas_call(
        _loss_body,
        out_shape=jax.ShapeDtypeStruct((1, 1), jnp.float32),
    )(q, c)
    return out[0, 0]


# speedup vs baseline: 1.8112x; 1.0016x over previous
import functools

import jax
import jax.numpy as jnp
import numpy as np
from jax.experimental import pallas as pl
from jax.experimental.pallas import tpu as pltpu

NUM_NEG = 128
MARGIN = 0.5
INT_MIN = np.int32(-2147483648)
INT_MAX = np.int32(2147483647)


def _f32_key(bits):
    return jnp.where(bits >= 0, bits, INT_MIN - bits)


def _colsum(x):
    h = x.shape[0] // 2
    while h >= 8:
        x = x[:h] + x[h:]
        h //= 2
    return jnp.sum(x, axis=0, keepdims=True)


def _loss_body(q_ref, c_ref, out_ref):
    q = q_ref[...]
    c = c_ref[...]
    lt = jax.lax.dot_general(
        c, q, (((1,), (1,)), ((), ())),
        preferred_element_type=jnp.float32,
        precision=jax.lax.Precision.HIGHEST,
    )

    C, B = lt.shape
    rows = jax.lax.broadcasted_iota(jnp.int32, (C, B), 0)
    cols = jax.lax.broadcasted_iota(jnp.int32, (C, B), 1)
    eye = rows == cols
    t = _colsum(jnp.where(eye, lt, 0.0))

    bits = jax.lax.bitcast_convert_type(lt, jnp.int32)
    key = _f32_key(bits)
    tkey = _f32_key(jax.lax.bitcast_convert_type(t, jnp.int32))
    neg = key < tkey
    mk = jnp.where(neg, key, INT_MIN)

    n = _colsum(neg.astype(jnp.int32))
    k = jnp.minimum(n, NUM_NEG)
    kk = jnp.maximum(k, 1)

    def _avg(a, b):
        return (a >> 1) + (b >> 1) + (a & b & 1)

    W2 = np.int32(1 << 11)
    W3 = np.int32(1 << 22)
    MASK11 = np.int32(0x7FF)

    def step(_, carry):
        lo, hi = carry
        m2 = _avg(lo, hi)
        m1 = _avg(lo, m2)
        m3 = _avg(m2, hi)
        e = jnp.where(mk >= m2,
                      jnp.where(mk >= m3, W3, W2),
                      jnp.where(mk >= m1, np.int32(1), np.int32(0)))
        tot = _colsum(e)
        c3 = jax.lax.shift_right_logical(tot, 22)
        c2 = c3 + (jax.lax.shift_right_logical(tot, 11) & MASK11)
        c1 = c2 + (tot & MASK11)
        p2 = c2 >= kk
        lo1 = jnp.where(p2, m2, lo)
        hi1 = jnp.where(p2, hi, m2)
        ps = jnp.where(p2, c3, c1) >= kk
        ms = jnp.where(p2, m3, m1)
        lo2 = jnp.where(ps, ms, lo1)
        hi2 = jnp.where(ps, hi1, ms)
        return lo2, hi2

    lo0 = jnp.full((1, B), INT_MIN, jnp.int32)
    hi0 = jnp.full((1, B), INT_MAX, jnp.int32)
    tau, _ = jax.lax.fori_loop(0, 16, step, (lo0, hi0))

    above = mk > tau
    c_gt = _colsum(above.astype(jnp.int32))
    m = (k - c_gt).astype(jnp.float32)
    tau_f = jax.lax.bitcast_convert_type(_f32_key(tau), jnp.float32)

    relu = jnp.maximum(lt - (1.0 - MARGIN), 0.0)
    num = _colsum(jnp.where(above, relu, 0.0))
    num = num + m * jnp.maximum(tau_f - (1.0 - MARGIN), 0.0)
    num = jnp.where(k > 0, num, 0.0)
    contrastive = num / (k.astype(jnp.float32) + 1e-9)

    align = jnp.sum(1.0 - t, axis=(0, 1), keepdims=True)
    out_ref[...] = align + jnp.sum(contrastive, axis=(0, 1), keepdims=True)


@jax.jit
def kernel(query_embed, candidate_embed):
    q = query_embed.reshape(query_embed.shape[0], query_embed.shape[2])
    c = candidate_embed.reshape(candidate_embed.shape[1],
                                candidate_embed.shape[2])
    out = pl.pallas_call(
        _loss_body,
        out_shape=jax.ShapeDtypeStruct((1, 1), jnp.float32),
    )(q, c)
    return out[0, 0]


# speedup vs baseline: 1.8620x; 1.0281x over previous
import functools

import jax
import jax.numpy as jnp
import numpy as np
from jax.experimental import pallas as pl
from jax.experimental.pallas import tpu as pltpu

NUM_NEG = 128
MARGIN = 0.5
INT_MIN = np.int32(-2147483648)
INT_MAX = np.int32(2147483647)


def _f32_key(bits):
    return jnp.where(bits >= 0, bits, INT_MIN - bits)


def _colsum(x):
    h = x.shape[0] // 2
    while h >= 8:
        x = x[:h] + x[h:]
        h //= 2
    return jnp.sum(x, axis=0, keepdims=True)


def _loss_body(q_ref, c_ref, out_ref):
    q = q_ref[...]
    c = c_ref[...]
    lt = jax.lax.dot_general(
        c, q, (((1,), (1,)), ((), ())),
        preferred_element_type=jnp.float32,
        precision=jax.lax.Precision.HIGHEST,
    )

    C, B = lt.shape
    rows = jax.lax.broadcasted_iota(jnp.int32, (C, B), 0)
    cols = jax.lax.broadcasted_iota(jnp.int32, (C, B), 1)
    eye = rows == cols
    t = _colsum(jnp.where(eye, lt, 0.0))

    bits = jax.lax.bitcast_convert_type(lt, jnp.int32)
    key = _f32_key(bits)
    tkey = _f32_key(jax.lax.bitcast_convert_type(t, jnp.int32))
    neg = key < tkey
    mk = jnp.where(neg, key, INT_MIN)

    n = _colsum(neg.astype(jnp.int32))
    k = jnp.minimum(n, NUM_NEG)
    kk = jnp.maximum(k, 1)

    def _avg(a, b):
        return (a >> 1) + (b >> 1) + (a & b & 1)

    W2 = np.int32(1 << 11)
    W3 = np.int32(1 << 22)
    MASK11 = np.int32(0x7FF)

    def cond(carry):
        lo, hi = carry
        return jnp.any((hi - lo) != 1)

    def step(carry):
        lo, hi = carry
        mid = _avg(lo, hi)
        cnt = _colsum((mk >= mid).astype(jnp.int32))
        pred = cnt >= kk
        return jnp.where(pred, mid, lo), jnp.where(pred, hi, mid)

    lo0 = jnp.full((1, B), INT_MIN, jnp.int32)
    hi0 = jnp.full((1, B), INT_MAX, jnp.int32)
    tau, _ = jax.lax.while_loop(cond, step, (lo0, hi0))

    above = mk > tau
    c_gt = _colsum(above.astype(jnp.int32))
    m = (k - c_gt).astype(jnp.float32)
    tau_f = jax.lax.bitcast_convert_type(_f32_key(tau), jnp.float32)

    relu = jnp.maximum(lt - (1.0 - MARGIN), 0.0)
    num = _colsum(jnp.where(above, relu, 0.0))
    num = num + m * jnp.maximum(tau_f - (1.0 - MARGIN), 0.0)
    num = jnp.where(k > 0, num, 0.0)
    contrastive = num / (k.astype(jnp.float32) + 1e-9)

    align = jnp.sum(1.0 - t, axis=(0, 1), keepdims=True)
    out_ref[...] = align + jnp.sum(contrastive, axis=(0, 1), keepdims=True)


@jax.jit
def kernel(query_embed, candidate_embed):
    q = query_embed.reshape(query_embed.shape[0], query_embed.shape[2])
    c = candidate_embed.reshape(candidate_embed.shape[1],
                                candidate_embed.shape[2])
    out = pl.pallas_call(
        _loss_body,
        out_shape=jax.ShapeDtypeStruct((1, 1), jnp.float32),
    )(q, c)
    return out[0, 0]
